# Initial kernel scaffold; baseline (speedup 1.0000x reference)
import jax
import jax.numpy as jnp
from jax.experimental import pallas as pl


def kernel(nbIp, edge_index, snap_feat, target_idx, W_gat, att_src, att_dst, gat_bias, mlp_w, mlp_b, ln_w, ln_b, W_ih, W_hh, b_ih, b_hh, l1_w, l1_b, l2_w, l2_b):
    raise NotImplementedError("write your pallas kernel here")


# speedup vs baseline: 1598.4734x; 1598.4734x over previous
import functools

import jax
import jax.numpy as jnp
from jax import lax
from jax.experimental import pallas as pl
from jax.experimental.pallas import tpu as pltpu
from jax.experimental.pallas import tpu_sc as plsc

B, T, NPG, EPG, HID, HEADS, FEAT = 8, 8, 1000, 16000, 64, 4, 16
G = B * T
N = G * NPG
E = G * EPG
OUTC = HID // HEADS
NCHUNK = EPG // 16
DEG_PAD = 1008
NEG_BIG = -1e30


def _leaky(x):
    return jnp.where(x > 0, x, 0.2 * x)


_sc_mesh = plsc.VectorSubcoreMesh(core_axis_name="c", subcore_axis_name="s")


@functools.partial(
    pl.kernel,
    mesh=_sc_mesh,
    out_type=jax.ShapeDtypeStruct((G, 16), jnp.float32),
    compiler_params=pltpu.CompilerParams(needs_layout_passes=False),
    scratch_types=[
        pltpu.VMEM((EPG,), jnp.int32),
        pltpu.VMEM((EPG,), jnp.int32),
        pltpu.VMEM((EPG + 16,), jnp.int32),
        pltpu.VMEM((DEG_PAD,), jnp.float32),
        pltpu.VMEM((NPG,), jnp.float32),
        pltpu.VMEM((G,), jnp.int32),
        pltpu.VMEM((2 * HID,), jnp.float32),
        pltpu.VMEM((HID,), jnp.float32),
        pltpu.VMEM((HID,), jnp.float32),
        pltpu.VMEM((16,), jnp.float32),
    ],
)
def _sc_gat(src_hbm, dst_hbm, nbip_hbm, tgt_hbm, wg_hbm, as_hbm, ad_hbm,
            out_hbm, src_v, dst_v, list_v, deg_v, nb_v, tgt_v, wg_v, as_v,
            ad_v, row_v):
    wid = lax.axis_index("s") * 2 + lax.axis_index("c")
    lanes = lax.broadcasted_iota(jnp.int32, (16,), 0)
    ones16 = jnp.ones((16,), jnp.float32)
    zeros16 = jnp.zeros((16,), jnp.float32)

    pltpu.sync_copy(tgt_hbm, tgt_v)
    pltpu.sync_copy(wg_hbm, wg_v)
    pltpu.sync_copy(as_hbm, as_v)
    pltpu.sync_copy(ad_hbm, ad_v)

    for k in range(2):
        g = wid * 2 + k
        base = g * NPG
        pltpu.sync_copy(src_hbm.at[pl.ds(g * EPG, EPG)], src_v)
        pltpu.sync_copy(dst_hbm.at[pl.ds(g * EPG, EPG)], dst_v)
        pltpu.sync_copy(nbip_hbm.at[pl.ds(base, NPG)], nb_v)

        for z in range(DEG_PAD // 16):
            deg_v[pl.ds(z * 16, 16)] = zeros16

        base_splat = jnp.full((16,), base, jnp.int32)
        tgt_splat = plsc.load_gather(tgt_v, [jnp.full((16,), g, jnp.int32)])

        def edge_body(i, cnt):
            s_loc = src_v[pl.ds(i * 16, 16)] - base_splat
            d = dst_v[pl.ds(i * 16, 16)]
            occ, last = plsc.scan_count(s_loc)
            plsc.addupdate_scatter(deg_v, [s_loc], occ.astype(jnp.float32),
                                   mask=last)
            msk = d == tgt_splat
            plsc.store_compressed(list_v.at[pl.ds(cnt, 16)], s_loc, mask=msk)
            return cnt + jnp.max(plsc.all_reduce_population_count(msk))

        cnt = lax.fori_loop(0, NCHUNK, edge_body, jnp.int32(0))

        t_loc = tgt_splat - base_splat
        nb_t = plsc.load_gather(nb_v, [t_loc])
        dg_t = plsc.load_gather(deg_v, [t_loc])
        nchunks = (cnt + 15) // 16

        row = zeros16
        for h in range(HEADS):
            w0h = wg_v[pl.ds(h * OUTC, 16)]
            w1h = wg_v[pl.ds(HID + h * OUTC, 16)]
            ash = as_v[pl.ds(h * OUTC, 16)]
            adh = ad_v[pl.ds(h * OUTC, 16)]
            c0 = jnp.sum(w0h * ash)
            c1 = jnp.sum(w1h * ash)
            d0 = jnp.sum(w0h * adh)
            d1 = jnp.sum(w1h * adh)
            a_d = nb_t * d0 + dg_t * d1
            a_self = _leaky(nb_t * c0 + dg_t * c1 + a_d)

            def alpha_at(j):
                valid = (lanes + j * 16) < cnt
                idx = jnp.where(valid, list_v[pl.ds(j * 16, 16)], 0)
                nb = plsc.load_gather(nb_v, [idx])
                dg = plsc.load_gather(deg_v, [idx])
                return valid, nb, dg, _leaky(nb * c0 + dg * c1 + a_d)

            def max_body(j, mx):
                valid, _, _, a = alpha_at(j)
                return jnp.maximum(mx, jnp.where(valid, a, NEG_BIG))

            m = jnp.max(lax.fori_loop(0, nchunks, max_body, a_self))

            def sum_body(j, acc):
                sa, sb, ss = acc
                valid, nb, dg, a = alpha_at(j)
                e = jnp.where(valid, jnp.exp(a - m), 0.0)
                return (sa + e * nb, sb + e * dg, ss + e)

            sa, sb, ss = lax.fori_loop(
                0, nchunks, sum_body, (zeros16, zeros16, zeros16))
            e_self = jnp.exp(a_self - m)
            S_v = e_self + jnp.sum(ss)
            A_v = (e_self * nb_t + jnp.sum(sa)) / S_v
            B_v = (e_self * dg_t + jnp.sum(sb)) / S_v
            row = jnp.where(lanes == h, A_v, row)
            row = jnp.where(lanes == (HEADS + h), B_v, row)

        row_v[...] = row
        r = (g % T) * B + (g // T)
        pltpu.sync_copy(row_v, out_hbm.at[r])


def _ln(x, w, b, eps=1e-5):
    mu = jnp.mean(x, axis=-1, keepdims=True)
    var = jnp.mean((x - mu) ** 2, axis=-1, keepdims=True)
    return (x - mu) * lax.rsqrt(var + eps) * w + b


def _tc_head(ab_ref, wg_ref, gb_ref, snap_ref, mlpw_ref, mlpb_ref, lnw_ref,
             lnb_ref, wir_ref, wiz_ref, win_ref, whr_ref, whz_ref, whn_ref,
             bir_ref, biz_ref, bin_ref, bhr_ref, bhz_ref, bhn_ref,
             l1w_ref, l1b_ref, l2w_ref, l2b_ref, out_ref):
    f32 = jnp.float32
    wg = wg_ref[...]
    rows = lax.broadcasted_iota(jnp.int32, (16, HID), 0)
    colh = lax.broadcasted_iota(jnp.int32, (16, HID), 1) // OUTC
    w0 = jnp.broadcast_to(wg[0:1, :], (16, HID))
    w1 = jnp.broadcast_to(wg[1:2, :], (16, HID))
    wm = jnp.where(rows == colh, w0, 0.0) + jnp.where(rows == colh + HEADS, w1, 0.0)
    h_node = jnp.maximum(
        jnp.dot(ab_ref[...], wm, preferred_element_type=f32) + gb_ref[...], 0.0)
    h_feat = _ln(
        jnp.maximum(
            jnp.dot(snap_ref[...], mlpw_ref[...], preferred_element_type=f32)
            + mlpb_ref[...], 0.0),
        lnw_ref[...], lnb_ref[...])
    fuse = _ln(h_node + h_feat, 1.0, 0.0)
    gi_r = jnp.dot(fuse, wir_ref[...], preferred_element_type=f32) + bir_ref[...]
    gi_z = jnp.dot(fuse, wiz_ref[...], preferred_element_type=f32) + biz_ref[...]
    gi_n = jnp.dot(fuse, win_ref[...], preferred_element_type=f32) + bin_ref[...]
    h = jnp.zeros((B, HID), f32)
    for t in range(T):
        sl = slice(t * B, (t + 1) * B)
        h_r = jnp.dot(h, whr_ref[...], preferred_element_type=f32) + bhr_ref[...]
        h_z = jnp.dot(h, whz_ref[...], preferred_element_type=f32) + bhz_ref[...]
        h_n = jnp.dot(h, whn_ref[...], preferred_element_type=f32) + bhn_ref[...]
        r = jax.nn.sigmoid(gi_r[sl, :] + h_r)
        z = jax.nn.sigmoid(gi_z[sl, :] + h_z)
        n = jnp.tanh(gi_n[sl, :] + r * h_n)
        h = (1.0 - z) * n + z * h
    o = jnp.maximum(
        jnp.dot(h, l1w_ref[...], preferred_element_type=f32) + l1b_ref[...], 0.0)
    out_ref[...] = jnp.dot(o, l2w_ref[...], preferred_element_type=f32) + l2b_ref[...]


def kernel(nbIp, edge_index, snap_feat, target_idx, W_gat, att_src, att_dst,
           gat_bias, mlp_w, mlp_b, ln_w, ln_b, W_ih, W_hh, b_ih, b_hh,
           l1_w, l1_b, l2_w, l2_b):
    src = edge_index[0]
    dst = edge_index[1]
    tgt = target_idx.reshape(-1)
    ab = _sc_gat(src, dst, nbIp, tgt, W_gat.reshape(-1), att_src.reshape(-1),
                 att_dst.reshape(-1))

    snap_t = jnp.transpose(snap_feat, (1, 0, 2)).reshape(G, FEAT)
    l2_pad = jnp.zeros((32, 128), jnp.float32).at[:, 0].set(l2_w[:, 0])
    l2b_pad = jnp.zeros((1, 128), jnp.float32).at[0, 0].set(l2_b[0])
    args = [
        ab, W_gat, gat_bias.reshape(1, HID), snap_t, mlp_w,
        mlp_b.reshape(1, HID), ln_w.reshape(1, HID), ln_b.reshape(1, HID),
        W_ih[0:HID, :].T, W_ih[HID:2 * HID, :].T, W_ih[2 * HID:, :].T,
        W_hh[0:HID, :].T, W_hh[HID:2 * HID, :].T, W_hh[2 * HID:, :].T,
        b_ih[0:HID].reshape(1, HID), b_ih[HID:2 * HID].reshape(1, HID),
        b_ih[2 * HID:].reshape(1, HID),
        b_hh[0:HID].reshape(1, HID), b_hh[HID:2 * HID].reshape(1, HID),
        b_hh[2 * HID:].reshape(1, HID),
        l1_w, l1_b.reshape(1, 32), l2_pad, l2b_pad,
    ]
    out = pl.pallas_call(
        _tc_head,
        out_shape=jax.ShapeDtypeStruct((B, 128), jnp.float32),
    )(*args)
    return out[:, 0:1]


# speedup vs baseline: 1609.7865x; 1.0071x over previous
import functools

import jax
import jax.numpy as jnp
from jax import lax
from jax.experimental import pallas as pl
from jax.experimental.pallas import tpu as pltpu
from jax.experimental.pallas import tpu_sc as plsc

B, T, NPG, EPG, HID, HEADS, FEAT = 8, 8, 1000, 16000, 64, 4, 16
G = B * T
N = G * NPG
E = G * EPG
OUTC = HID // HEADS
NCHUNK = EPG // 16
DEG_PAD = 1008
NEG_BIG = -1e30


def _leaky(x):
    return jnp.where(x > 0, x, 0.2 * x)


def _round_bf16_f32(x):
    u = plsc.bitcast(x, jnp.int32)
    r = u + jnp.int32(0x7FFF) + (lax.shift_right_logical(u, 16) & 1)
    return plsc.bitcast(r & jnp.int32(-65536), jnp.float32)


_sc_mesh = plsc.VectorSubcoreMesh(core_axis_name="c", subcore_axis_name="s")


@functools.partial(
    pl.kernel,
    mesh=_sc_mesh,
    out_type=jax.ShapeDtypeStruct((G, 16), jnp.float32),
    compiler_params=pltpu.CompilerParams(needs_layout_passes=False),
    scratch_types=[
        pltpu.VMEM((EPG,), jnp.int32),
        pltpu.VMEM((EPG,), jnp.int32),
        pltpu.VMEM((EPG + 16,), jnp.int32),
        pltpu.VMEM((DEG_PAD,), jnp.float32),
        pltpu.VMEM((NPG,), jnp.float32),
        pltpu.VMEM((G,), jnp.int32),
        pltpu.VMEM((2 * HID,), jnp.float32),
        pltpu.VMEM((HID,), jnp.float32),
        pltpu.VMEM((HID,), jnp.float32),
        pltpu.VMEM((16,), jnp.float32),
    ],
)
def _sc_gat(src_hbm, dst_hbm, nbip_hbm, tgt_hbm, wg_hbm, as_hbm, ad_hbm,
            out_hbm, src_v, dst_v, list_v, deg_v, nb_v, tgt_v, wg_v, as_v,
            ad_v, row_v):
    wid = lax.axis_index("s") * 2 + lax.axis_index("c")
    lanes = lax.broadcasted_iota(jnp.int32, (16,), 0)
    ones16 = jnp.ones((16,), jnp.float32)
    zeros16 = jnp.zeros((16,), jnp.float32)

    pltpu.sync_copy(tgt_hbm, tgt_v)
    pltpu.sync_copy(wg_hbm, wg_v)
    pltpu.sync_copy(as_hbm, as_v)
    pltpu.sync_copy(ad_hbm, ad_v)
    for z in range(2 * HID // 16):
        wg_v[pl.ds(z * 16, 16)] = _round_bf16_f32(wg_v[pl.ds(z * 16, 16)])

    for k in range(2):
        g = wid * 2 + k
        base = g * NPG
        pltpu.sync_copy(src_hbm.at[pl.ds(g * EPG, EPG)], src_v)
        pltpu.sync_copy(dst_hbm.at[pl.ds(g * EPG, EPG)], dst_v)
        pltpu.sync_copy(nbip_hbm.at[pl.ds(base, NPG)], nb_v)

        for z in range(DEG_PAD // 16):
            deg_v[pl.ds(z * 16, 16)] = zeros16

        base_splat = jnp.full((16,), base, jnp.int32)
        tgt_splat = plsc.load_gather(tgt_v, [jnp.full((16,), g, jnp.int32)])

        def edge_body(i, cnt_v):
            s_loc = src_v[pl.ds(i * 16, 16)] - base_splat
            d = dst_v[pl.ds(i * 16, 16)]
            occ, last = plsc.scan_count(s_loc)
            plsc.addupdate_scatter(deg_v, [s_loc], occ.astype(jnp.float32),
                                   mask=last)
            msk = d == tgt_splat
            pre = plsc.cumsum(msk.astype(jnp.int32))
            plsc.store_scatter(list_v, [cnt_v + pre - 1], s_loc, mask=msk)
            return cnt_v + plsc.all_reduce_population_count(msk)

        cnt_v = lax.fori_loop(0, NCHUNK, edge_body,
                              jnp.zeros((16,), jnp.int32))
        cnt = jnp.max(cnt_v)

        for z in range(62):
            nb_v[pl.ds(z * 16, 16)] = _round_bf16_f32(nb_v[pl.ds(z * 16, 16)])
        nb_v[pl.ds(NPG - 16, 16)] = _round_bf16_f32(nb_v[pl.ds(NPG - 16, 16)])
        for z in range(DEG_PAD // 16):
            deg_v[pl.ds(z * 16, 16)] = _round_bf16_f32(
                deg_v[pl.ds(z * 16, 16)])

        t_loc = tgt_splat - base_splat
        nb_t = plsc.load_gather(nb_v, [t_loc])
        dg_t = plsc.load_gather(deg_v, [t_loc])
        nchunks = (cnt + 15) // 16

        row = zeros16
        for h in range(HEADS):
            w0h = wg_v[pl.ds(h * OUTC, 16)]
            w1h = wg_v[pl.ds(HID + h * OUTC, 16)]
            ash = as_v[pl.ds(h * OUTC, 16)]
            adh = ad_v[pl.ds(h * OUTC, 16)]
            c0 = jnp.sum(w0h * ash)
            c1 = jnp.sum(w1h * ash)
            d0 = jnp.sum(w0h * adh)
            d1 = jnp.sum(w1h * adh)
            a_d = nb_t * d0 + dg_t * d1
            a_self = _leaky(nb_t * c0 + dg_t * c1 + a_d)

            def alpha_at(j):
                valid = (lanes + j * 16) < cnt
                idx = jnp.where(valid, list_v[pl.ds(j * 16, 16)], 0)
                nb = plsc.load_gather(nb_v, [idx])
                dg = plsc.load_gather(deg_v, [idx])
                return valid, nb, dg, _leaky(nb * c0 + dg * c1 + a_d)

            def max_body(j, mx):
                valid, _, _, a = alpha_at(j)
                return jnp.maximum(mx, jnp.where(valid, a, NEG_BIG))

            m = jnp.max(lax.fori_loop(0, nchunks, max_body, a_self))

            def sum_body(j, acc):
                sa, sb, ss = acc
                valid, nb, dg, a = alpha_at(j)
                e = jnp.where(valid, jnp.exp(a - m), 0.0)
                return (sa + e * nb, sb + e * dg, ss + e)

            sa, sb, ss = lax.fori_loop(
                0, nchunks, sum_body, (zeros16, zeros16, zeros16))
            e_self = jnp.exp(a_self - m)
            S_v = e_self + jnp.sum(ss)
            A_v = (e_self * nb_t + jnp.sum(sa)) / S_v
            B_v = (e_self * dg_t + jnp.sum(sb)) / S_v
            row = jnp.where(lanes == h, A_v, row)
            row = jnp.where(lanes == (HEADS + h), B_v, row)

        row_v[...] = row
        r = (g % T) * B + (g // T)
        pltpu.sync_copy(row_v, out_hbm.at[r])


def _ln(x, w, b, eps=1e-5):
    mu = jnp.mean(x, axis=-1, keepdims=True)
    var = jnp.mean((x - mu) ** 2, axis=-1, keepdims=True)
    return (x - mu) / jnp.sqrt(var + eps) * w + b


def _dot_bf16(a, b):
    return jnp.dot(a.astype(jnp.bfloat16), b.astype(jnp.bfloat16),
                   preferred_element_type=jnp.float32)


def _tc_head(ab_ref, wg_ref, gb_ref, snap_ref, mlpw_ref, mlpb_ref, lnw_ref,
             lnb_ref, wir_ref, wiz_ref, win_ref, whr_ref, whz_ref, whn_ref,
             bir_ref, biz_ref, bin_ref, bhr_ref, bhz_ref, bhn_ref,
             l1w_ref, l1b_ref, l2w_ref, l2b_ref, out_ref):
    f32 = jnp.float32
    wg = wg_ref[...]
    rows = lax.broadcasted_iota(jnp.int32, (16, HID), 0)
    colh = lax.broadcasted_iota(jnp.int32, (16, HID), 1) // OUTC
    sel_a = jnp.where(rows == colh, 1.0, 0.0).astype(f32)
    sel_b = jnp.where(rows == colh + HEADS, 1.0, 0.0).astype(f32)
    ab = ab_ref[...]
    a_exp = jnp.dot(ab, sel_a, preferred_element_type=f32,
                    precision=lax.Precision.HIGHEST)
    b_exp = jnp.dot(ab, sel_b, preferred_element_type=f32,
                    precision=lax.Precision.HIGHEST)
    wgb = wg.astype(jnp.bfloat16).astype(f32)
    w0 = jnp.broadcast_to(wgb[0:1, :], (G, HID))
    w1 = jnp.broadcast_to(wgb[1:2, :], (G, HID))
    h_node = jnp.maximum(a_exp * w0 + b_exp * w1 + gb_ref[...], 0.0)
    h_feat = _ln(
        jnp.maximum(_dot_bf16(snap_ref[...], mlpw_ref[...]) + mlpb_ref[...],
                    0.0),
        lnw_ref[...], lnb_ref[...])
    fuse = _ln(h_node + h_feat, 1.0, 0.0)
    gi_r = _dot_bf16(fuse, wir_ref[...]) + bir_ref[...]
    gi_z = _dot_bf16(fuse, wiz_ref[...]) + biz_ref[...]
    gi_n = _dot_bf16(fuse, win_ref[...]) + bin_ref[...]
    h = jnp.zeros((B, HID), f32)
    for t in range(T):
        sl = slice(t * B, (t + 1) * B)
        h_r = _dot_bf16(h, whr_ref[...]) + bhr_ref[...]
        h_z = _dot_bf16(h, whz_ref[...]) + bhz_ref[...]
        h_n = _dot_bf16(h, whn_ref[...]) + bhn_ref[...]
        r = jax.nn.sigmoid(gi_r[sl, :] + h_r)
        z = jax.nn.sigmoid(gi_z[sl, :] + h_z)
        n = jnp.tanh(gi_n[sl, :] + r * h_n)
        h = (1.0 - z) * n + z * h
    o = jnp.maximum(_dot_bf16(h, l1w_ref[...]) + l1b_ref[...], 0.0)
    out_ref[...] = _dot_bf16(o, l2w_ref[...]) + l2b_ref[...]


def kernel(nbIp, edge_index, snap_feat, target_idx, W_gat, att_src, att_dst,
           gat_bias, mlp_w, mlp_b, ln_w, ln_b, W_ih, W_hh, b_ih, b_hh,
           l1_w, l1_b, l2_w, l2_b):
    src = edge_index[0]
    dst = edge_index[1]
    tgt = target_idx.reshape(-1)
    ab = _sc_gat(src, dst, nbIp, tgt, W_gat.reshape(-1), att_src.reshape(-1),
                 att_dst.reshape(-1))

    snap_t = jnp.transpose(snap_feat, (1, 0, 2)).reshape(G, FEAT)
    l2_pad = jnp.zeros((32, 128), jnp.float32).at[:, 0].set(l2_w[:, 0])
    l2b_pad = jnp.zeros((1, 128), jnp.float32).at[0, 0].set(l2_b[0])
    args = [
        ab, W_gat, gat_bias.reshape(1, HID), snap_t, mlp_w,
        mlp_b.reshape(1, HID), ln_w.reshape(1, HID), ln_b.reshape(1, HID),
        W_ih[0:HID, :].T, W_ih[HID:2 * HID, :].T, W_ih[2 * HID:, :].T,
        W_hh[0:HID, :].T, W_hh[HID:2 * HID, :].T, W_hh[2 * HID:, :].T,
        b_ih[0:HID].reshape(1, HID), b_ih[HID:2 * HID].reshape(1, HID),
        b_ih[2 * HID:].reshape(1, HID),
        b_hh[0:HID].reshape(1, HID), b_hh[HID:2 * HID].reshape(1, HID),
        b_hh[2 * HID:].reshape(1, HID),
        l1_w, l1_b.reshape(1, 32), l2_pad, l2b_pad,
    ]
    out = pl.pallas_call(
        _tc_head,
        out_shape=jax.ShapeDtypeStruct((B, 128), jnp.float32),
    )(*args)
    return out[:, 0:1]


# speedup vs baseline: 1709.5981x; 1.0620x over previous
import functools

import jax
import jax.numpy as jnp
from jax import lax
from jax.experimental import pallas as pl
from jax.experimental.pallas import tpu as pltpu
from jax.experimental.pallas import tpu_sc as plsc

B, T, NPG, EPG, HID, HEADS, FEAT = 8, 8, 1000, 16000, 64, 4, 16
G = B * T
N = G * NPG
E = G * EPG
OUTC = HID // HEADS
NCHUNK = EPG // 16
DEG_PAD = 1008
NEG_BIG = -1e30


def _leaky(x):
    return jnp.where(x > 0, x, 0.2 * x)


def _round_bf16_f32(x):
    u = plsc.bitcast(x, jnp.int32)
    r = u + jnp.int32(0x7FFF) + (lax.shift_right_logical(u, 16) & 1)
    return plsc.bitcast(r & jnp.int32(-65536), jnp.float32)


_sc_mesh = plsc.VectorSubcoreMesh(core_axis_name="c", subcore_axis_name="s")


@functools.partial(
    pl.kernel,
    mesh=_sc_mesh,
    out_type=jax.ShapeDtypeStruct((G, 16), jnp.float32),
    compiler_params=pltpu.CompilerParams(needs_layout_passes=False),
    scratch_types=[
        pltpu.VMEM((EPG,), jnp.int32),
        pltpu.VMEM((EPG,), jnp.int32),
        pltpu.VMEM((EPG,), jnp.int32),
        pltpu.VMEM((EPG,), jnp.int32),
        pltpu.VMEM((EPG + 16,), jnp.int32),
        pltpu.VMEM((DEG_PAD,), jnp.float32),
        pltpu.VMEM((NPG,), jnp.float32),
        pltpu.VMEM((NPG,), jnp.float32),
        pltpu.VMEM((G,), jnp.int32),
        pltpu.VMEM((2 * HID,), jnp.float32),
        pltpu.VMEM((HID,), jnp.float32),
        pltpu.VMEM((HID,), jnp.float32),
        pltpu.VMEM((16,), jnp.float32),
        pltpu.SemaphoreType.DMA,
        pltpu.SemaphoreType.DMA,
    ],
)
def _sc_gat(src_hbm, dst_hbm, nbip_hbm, tgt_hbm, wg_hbm, as_hbm, ad_hbm,
            out_hbm, src_a, dst_a, src_b, dst_b, list_v, deg_v, nb_a, nb_b,
            tgt_v, wg_v, as_v, ad_v, row_v, sem_a, sem_b):
    wid = lax.axis_index("s") * 2 + lax.axis_index("c")
    lanes = lax.broadcasted_iota(jnp.int32, (16,), 0)
    zeros16 = jnp.zeros((16,), jnp.float32)

    g_a = wid * 2
    g_b = wid * 2 + 1
    cps = []
    for g, sv, dv, nv, sem in ((g_a, src_a, dst_a, nb_a, sem_a),
                               (g_b, src_b, dst_b, nb_b, sem_b)):
        cps.append((
            pltpu.async_copy(src_hbm.at[pl.ds(g * EPG, EPG)], sv, sem),
            pltpu.async_copy(dst_hbm.at[pl.ds(g * EPG, EPG)], dv, sem),
            pltpu.async_copy(nbip_hbm.at[pl.ds(g * NPG, NPG)], nv, sem),
        ))

    pltpu.sync_copy(tgt_hbm, tgt_v)
    pltpu.sync_copy(wg_hbm, wg_v)
    pltpu.sync_copy(as_hbm, as_v)
    pltpu.sync_copy(ad_hbm, ad_v)
    for z in range(2 * HID // 16):
        wg_v[pl.ds(z * 16, 16)] = _round_bf16_f32(wg_v[pl.ds(z * 16, 16)])

    for k, (g, src_v, dst_v, nb_v) in enumerate(
            ((g_a, src_a, dst_a, nb_a), (g_b, src_b, dst_b, nb_b))):
        base = g * NPG
        for cp in cps[k]:
            cp.wait()

        for z in range(DEG_PAD // 16):
            deg_v[pl.ds(z * 16, 16)] = zeros16

        base_splat = jnp.full((16,), base, jnp.int32)
        tgt_splat = plsc.load_gather(tgt_v, [jnp.full((16,), g, jnp.int32)])

        UNROLL = 8

        def edge_body(i, cnt_v):
            for u in range(UNROLL):
                off = i * (16 * UNROLL) + u * 16
                s_loc = src_v[pl.ds(off, 16)] - base_splat
                d = dst_v[pl.ds(off, 16)]
                occ, last = plsc.scan_count(s_loc)
                plsc.addupdate_scatter(deg_v, [s_loc],
                                       occ.astype(jnp.float32), mask=last)
                msk = d == tgt_splat
                pre = plsc.cumsum(msk.astype(jnp.int32))
                plsc.store_scatter(list_v, [cnt_v + pre - 1], s_loc,
                                   mask=msk)
                cnt_v = cnt_v + plsc.all_reduce_population_count(msk)
            return cnt_v

        cnt_v = lax.fori_loop(0, NCHUNK // UNROLL, edge_body,
                              jnp.zeros((16,), jnp.int32))
        cnt = jnp.max(cnt_v)

        for z in range(62):
            nb_v[pl.ds(z * 16, 16)] = _round_bf16_f32(nb_v[pl.ds(z * 16, 16)])
        nb_v[pl.ds(NPG - 16, 16)] = _round_bf16_f32(nb_v[pl.ds(NPG - 16, 16)])
        for z in range(DEG_PAD // 16):
            deg_v[pl.ds(z * 16, 16)] = _round_bf16_f32(
                deg_v[pl.ds(z * 16, 16)])

        t_loc = tgt_splat - base_splat
        nb_t = plsc.load_gather(nb_v, [t_loc])
        dg_t = plsc.load_gather(deg_v, [t_loc])
        nchunks = (cnt + 15) // 16

        row = zeros16
        for h in range(HEADS):
            w0h = wg_v[pl.ds(h * OUTC, 16)]
            w1h = wg_v[pl.ds(HID + h * OUTC, 16)]
            ash = as_v[pl.ds(h * OUTC, 16)]
            adh = ad_v[pl.ds(h * OUTC, 16)]
            c0 = jnp.sum(w0h * ash)
            c1 = jnp.sum(w1h * ash)
            d0 = jnp.sum(w0h * adh)
            d1 = jnp.sum(w1h * adh)
            a_d = nb_t * d0 + dg_t * d1
            a_self = _leaky(nb_t * c0 + dg_t * c1 + a_d)

            def alpha_at(j):
                valid = (lanes + j * 16) < cnt
                idx = jnp.where(valid, list_v[pl.ds(j * 16, 16)], 0)
                nb = plsc.load_gather(nb_v, [idx])
                dg = plsc.load_gather(deg_v, [idx])
                return valid, nb, dg, _leaky(nb * c0 + dg * c1 + a_d)

            def max_body(j, mx):
                valid, _, _, a = alpha_at(j)
                return jnp.maximum(mx, jnp.where(valid, a, NEG_BIG))

            m = jnp.max(lax.fori_loop(0, nchunks, max_body, a_self))

            def sum_body(j, acc):
                sa, sb, ss = acc
                valid, nb, dg, a = alpha_at(j)
                e = jnp.where(valid, jnp.exp(a - m), 0.0)
                return (sa + e * nb, sb + e * dg, ss + e)

            sa, sb, ss = lax.fori_loop(
                0, nchunks, sum_body, (zeros16, zeros16, zeros16))
            e_self = jnp.exp(a_self - m)
            S_v = e_self + jnp.sum(ss)
            A_v = (e_self * nb_t + jnp.sum(sa)) / S_v
            B_v = (e_self * dg_t + jnp.sum(sb)) / S_v
            row = jnp.where(lanes == h, A_v, row)
            row = jnp.where(lanes == (HEADS + h), B_v, row)

        row_v[...] = row
        r = (g % T) * B + (g // T)
        pltpu.sync_copy(row_v, out_hbm.at[r])


def _ln(x, w, b, eps=1e-5):
    mu = jnp.mean(x, axis=-1, keepdims=True)
    var = jnp.mean((x - mu) ** 2, axis=-1, keepdims=True)
    return (x - mu) / jnp.sqrt(var + eps) * w + b


def _dot_bf16(a, b):
    return jnp.dot(a.astype(jnp.bfloat16), b.astype(jnp.bfloat16),
                   preferred_element_type=jnp.float32)


def _tc_head(ab_ref, wg_ref, gb_ref, snap_ref, mlpw_ref, mlpb_ref, lnw_ref,
             lnb_ref, wir_ref, wiz_ref, win_ref, whr_ref, whz_ref, whn_ref,
             bir_ref, biz_ref, bin_ref, bhr_ref, bhz_ref, bhn_ref,
             l1w_ref, l1b_ref, l2w_ref, l2b_ref, out_ref):
    f32 = jnp.float32
    wg = wg_ref[...]
    rows = lax.broadcasted_iota(jnp.int32, (16, HID), 0)
    colh = lax.broadcasted_iota(jnp.int32, (16, HID), 1) // OUTC
    sel_a = jnp.where(rows == colh, 1.0, 0.0).astype(f32)
    sel_b = jnp.where(rows == colh + HEADS, 1.0, 0.0).astype(f32)
    ab = ab_ref[...]
    a_exp = jnp.dot(ab, sel_a, preferred_element_type=f32,
                    precision=lax.Precision.HIGHEST)
    b_exp = jnp.dot(ab, sel_b, preferred_element_type=f32,
                    precision=lax.Precision.HIGHEST)
    wgb = wg.astype(jnp.bfloat16).astype(f32)
    w0 = jnp.broadcast_to(wgb[0:1, :], (G, HID))
    w1 = jnp.broadcast_to(wgb[1:2, :], (G, HID))
    h_node = jnp.maximum(a_exp * w0 + b_exp * w1 + gb_ref[...], 0.0)
    h_feat = _ln(
        jnp.maximum(_dot_bf16(snap_ref[...], mlpw_ref[...]) + mlpb_ref[...],
                    0.0),
        lnw_ref[...], lnb_ref[...])
    fuse = _ln(h_node + h_feat, 1.0, 0.0)
    gi_r = _dot_bf16(fuse, wir_ref[...]) + bir_ref[...]
    gi_z = _dot_bf16(fuse, wiz_ref[...]) + biz_ref[...]
    gi_n = _dot_bf16(fuse, win_ref[...]) + bin_ref[...]
    h = jnp.zeros((B, HID), f32)
    for t in range(T):
        sl = slice(t * B, (t + 1) * B)
        h_r = _dot_bf16(h, whr_ref[...]) + bhr_ref[...]
        h_z = _dot_bf16(h, whz_ref[...]) + bhz_ref[...]
        h_n = _dot_bf16(h, whn_ref[...]) + bhn_ref[...]
        r = jax.nn.sigmoid(gi_r[sl, :] + h_r)
        z = jax.nn.sigmoid(gi_z[sl, :] + h_z)
        n = jnp.tanh(gi_n[sl, :] + r * h_n)
        h = (1.0 - z) * n + z * h
    o = jnp.maximum(_dot_bf16(h, l1w_ref[...]) + l1b_ref[...], 0.0)
    out_ref[...] = _dot_bf16(o, l2w_ref[...]) + l2b_ref[...]


def kernel(nbIp, edge_index, snap_feat, target_idx, W_gat, att_src, att_dst,
           gat_bias, mlp_w, mlp_b, ln_w, ln_b, W_ih, W_hh, b_ih, b_hh,
           l1_w, l1_b, l2_w, l2_b):
    src = edge_index[0]
    dst = edge_index[1]
    tgt = target_idx.reshape(-1)
    ab = _sc_gat(src, dst, nbIp, tgt, W_gat.reshape(-1), att_src.reshape(-1),
                 att_dst.reshape(-1))

    snap_t = jnp.transpose(snap_feat, (1, 0, 2)).reshape(G, FEAT)
    l2_pad = jnp.zeros((32, 128), jnp.float32).at[:, 0].set(l2_w[:, 0])
    l2b_pad = jnp.zeros((1, 128), jnp.float32).at[0, 0].set(l2_b[0])
    args = [
        ab, W_gat, gat_bias.reshape(1, HID), snap_t, mlp_w,
        mlp_b.reshape(1, HID), ln_w.reshape(1, HID), ln_b.reshape(1, HID),
        W_ih[0:HID, :].T, W_ih[HID:2 * HID, :].T, W_ih[2 * HID:, :].T,
        W_hh[0:HID, :].T, W_hh[HID:2 * HID, :].T, W_hh[2 * HID:, :].T,
        b_ih[0:HID].reshape(1, HID), b_ih[HID:2 * HID].reshape(1, HID),
        b_ih[2 * HID:].reshape(1, HID),
        b_hh[0:HID].reshape(1, HID), b_hh[HID:2 * HID].reshape(1, HID),
        b_hh[2 * HID:].reshape(1, HID),
        l1_w, l1_b.reshape(1, 32), l2_pad, l2b_pad,
    ]
    out = pl.pallas_call(
        _tc_head,
        out_shape=jax.ShapeDtypeStruct((B, 128), jnp.float32),
    )(*args)
    return out[:, 0:1]


# speedup vs baseline: 2605.1508x; 1.5238x over previous
import functools

import jax
import jax.numpy as jnp
from jax import lax
from jax.experimental import pallas as pl
from jax.experimental.pallas import tpu as pltpu
from jax.experimental.pallas import tpu_sc as plsc

B, T, NPG, EPG, HID, HEADS, FEAT = 8, 8, 1000, 16000, 64, 4, 16
G = B * T
N = G * NPG
E = G * EPG
OUTC = HID // HEADS
NCHUNK = EPG // 16
DEG_PAD = 1008
NEG_BIG = -1e30


def _leaky(x):
    return jnp.where(x > 0, x, 0.2 * x)


def _round_bf16_f32(x):
    u = plsc.bitcast(x, jnp.int32)
    r = u + jnp.int32(0x7FFF) + (lax.shift_right_logical(u, 16) & 1)
    return plsc.bitcast(r & jnp.int32(-65536), jnp.float32)


_sc_mesh = plsc.VectorSubcoreMesh(core_axis_name="c", subcore_axis_name="s")


@functools.partial(
    pl.kernel,
    mesh=_sc_mesh,
    out_type=jax.ShapeDtypeStruct((G, 16), jnp.float32),
    compiler_params=pltpu.CompilerParams(needs_layout_passes=False),
    scratch_types=[
        pltpu.VMEM((EPG,), jnp.int32),
        pltpu.VMEM((EPG,), jnp.int32),
        pltpu.VMEM((EPG,), jnp.int32),
        pltpu.VMEM((EPG,), jnp.int32),
        pltpu.VMEM((EPG + 16,), jnp.int32),
        pltpu.VMEM((DEG_PAD,), jnp.float32),
        pltpu.VMEM((NPG,), jnp.float32),
        pltpu.VMEM((NPG,), jnp.float32),
        pltpu.VMEM((G,), jnp.int32),
        pltpu.VMEM((2 * HID,), jnp.float32),
        pltpu.VMEM((HID,), jnp.float32),
        pltpu.VMEM((HID,), jnp.float32),
        pltpu.VMEM((16,), jnp.float32),
        pltpu.SemaphoreType.DMA,
        pltpu.SemaphoreType.DMA,
    ],
)
def _sc_gat(src_hbm, dst_hbm, nbip_hbm, tgt_hbm, wg_hbm, as_hbm, ad_hbm,
            out_hbm, src_a, dst_a, src_b, dst_b, list_v, deg_v, nb_a, nb_b,
            tgt_v, wg_v, as_v, ad_v, row_v, sem_a, sem_b):
    wid = lax.axis_index("s") * 2 + lax.axis_index("c")
    lanes = lax.broadcasted_iota(jnp.int32, (16,), 0)
    zeros16 = jnp.zeros((16,), jnp.float32)

    g_a = wid * 2
    g_b = wid * 2 + 1
    cps = []
    for g, sv, dv, nv, sem in ((g_a, src_a, dst_a, nb_a, sem_a),
                               (g_b, src_b, dst_b, nb_b, sem_b)):
        cps.append((
            pltpu.async_copy(src_hbm.at[pl.ds(g * EPG, EPG)], sv, sem),
            pltpu.async_copy(dst_hbm.at[pl.ds(g * EPG, EPG)], dv, sem),
            pltpu.async_copy(nbip_hbm.at[pl.ds(g * NPG, NPG)], nv, sem),
        ))

    pltpu.sync_copy(tgt_hbm, tgt_v)
    pltpu.sync_copy(wg_hbm, wg_v)
    pltpu.sync_copy(as_hbm, as_v)
    pltpu.sync_copy(ad_hbm, ad_v)
    for z in range(2 * HID // 16):
        wg_v[pl.ds(z * 16, 16)] = _round_bf16_f32(wg_v[pl.ds(z * 16, 16)])

    for k, (g, src_v, dst_v, nb_v) in enumerate(
            ((g_a, src_a, dst_a, nb_a), (g_b, src_b, dst_b, nb_b))):
        base = g * NPG
        for cp in cps[k]:
            cp.wait()

        for z in range(DEG_PAD // 16):
            deg_v[pl.ds(z * 16, 16)] = zeros16

        base_splat = jnp.full((16,), base, jnp.int32)
        tgt_splat = plsc.load_gather(tgt_v, [jnp.full((16,), g, jnp.int32)])

        def edge_body(i, cnt_v):
            off = i * 16
            s_loc = src_v[pl.ds(off, 16)] - base_splat
            d = dst_v[pl.ds(off, 16)]
            occ, last = plsc.scan_count(s_loc)
            plsc.addupdate_scatter(deg_v, [s_loc],
                                   occ.astype(jnp.float32), mask=last)
            msk = d == tgt_splat
            pre = plsc.cumsum(msk.astype(jnp.int32))
            plsc.store_scatter(list_v, [cnt_v + pre - 1], s_loc, mask=msk)
            return cnt_v + plsc.all_reduce_population_count(msk)

        cnt_v = plsc.parallel_loop(
            0, NCHUNK, 1, unroll=8,
            carry=jnp.zeros((16,), jnp.int32))(edge_body)
        cnt = jnp.max(cnt_v)

        for z in range(62):
            nb_v[pl.ds(z * 16, 16)] = _round_bf16_f32(nb_v[pl.ds(z * 16, 16)])
        nb_v[pl.ds(NPG - 16, 16)] = _round_bf16_f32(nb_v[pl.ds(NPG - 16, 16)])
        for z in range(DEG_PAD // 16):
            deg_v[pl.ds(z * 16, 16)] = _round_bf16_f32(
                deg_v[pl.ds(z * 16, 16)])

        t_loc = tgt_splat - base_splat
        nb_t = plsc.load_gather(nb_v, [t_loc])
        dg_t = plsc.load_gather(deg_v, [t_loc])
        nchunks = (cnt + 15) // 16

        row = zeros16
        for h in range(HEADS):
            w0h = wg_v[pl.ds(h * OUTC, 16)]
            w1h = wg_v[pl.ds(HID + h * OUTC, 16)]
            ash = as_v[pl.ds(h * OUTC, 16)]
            adh = ad_v[pl.ds(h * OUTC, 16)]
            c0 = jnp.sum(w0h * ash)
            c1 = jnp.sum(w1h * ash)
            d0 = jnp.sum(w0h * adh)
            d1 = jnp.sum(w1h * adh)
            a_d = nb_t * d0 + dg_t * d1
            a_self = _leaky(nb_t * c0 + dg_t * c1 + a_d)

            def alpha_at(j):
                valid = (lanes + j * 16) < cnt
                idx = jnp.where(valid, list_v[pl.ds(j * 16, 16)], 0)
                nb = plsc.load_gather(nb_v, [idx])
                dg = plsc.load_gather(deg_v, [idx])
                return valid, nb, dg, _leaky(nb * c0 + dg * c1 + a_d)

            def max_body(j, mx):
                valid, _, _, a = alpha_at(j)
                return jnp.maximum(mx, jnp.where(valid, a, NEG_BIG))

            m = jnp.max(lax.fori_loop(0, nchunks, max_body, a_self))

            def sum_body(j, acc):
                sa, sb, ss = acc
                valid, nb, dg, a = alpha_at(j)
                e = jnp.where(valid, jnp.exp(a - m), 0.0)
                return (sa + e * nb, sb + e * dg, ss + e)

            sa, sb, ss = lax.fori_loop(
                0, nchunks, sum_body, (zeros16, zeros16, zeros16))
            e_self = jnp.exp(a_self - m)
            S_v = e_self + jnp.sum(ss)
            A_v = (e_self * nb_t + jnp.sum(sa)) / S_v
            B_v = (e_self * dg_t + jnp.sum(sb)) / S_v
            row = jnp.where(lanes == h, A_v, row)
            row = jnp.where(lanes == (HEADS + h), B_v, row)

        row_v[...] = row
        r = (g % T) * B + (g // T)
        pltpu.sync_copy(row_v, out_hbm.at[r])


def _ln(x, w, b, eps=1e-5):
    mu = jnp.mean(x, axis=-1, keepdims=True)
    var = jnp.mean((x - mu) ** 2, axis=-1, keepdims=True)
    return (x - mu) / jnp.sqrt(var + eps) * w + b


def _dot_bf16(a, b):
    return jnp.dot(a.astype(jnp.bfloat16), b.astype(jnp.bfloat16),
                   preferred_element_type=jnp.float32)


def _tc_head(ab_ref, wg_ref, gb_ref, snap_ref, mlpw_ref, mlpb_ref, lnw_ref,
             lnb_ref, wir_ref, wiz_ref, win_ref, whr_ref, whz_ref, whn_ref,
             bir_ref, biz_ref, bin_ref, bhr_ref, bhz_ref, bhn_ref,
             l1w_ref, l1b_ref, l2w_ref, l2b_ref, out_ref):
    f32 = jnp.float32
    wg = wg_ref[...]
    rows = lax.broadcasted_iota(jnp.int32, (16, HID), 0)
    colh = lax.broadcasted_iota(jnp.int32, (16, HID), 1) // OUTC
    sel_a = jnp.where(rows == colh, 1.0, 0.0).astype(f32)
    sel_b = jnp.where(rows == colh + HEADS, 1.0, 0.0).astype(f32)
    ab = ab_ref[...]
    a_exp = jnp.dot(ab, sel_a, preferred_element_type=f32,
                    precision=lax.Precision.HIGHEST)
    b_exp = jnp.dot(ab, sel_b, preferred_element_type=f32,
                    precision=lax.Precision.HIGHEST)
    wgb = wg.astype(jnp.bfloat16).astype(f32)
    w0 = jnp.broadcast_to(wgb[0:1, :], (G, HID))
    w1 = jnp.broadcast_to(wgb[1:2, :], (G, HID))
    h_node = jnp.maximum(a_exp * w0 + b_exp * w1 + gb_ref[...], 0.0)
    h_feat = _ln(
        jnp.maximum(_dot_bf16(snap_ref[...], mlpw_ref[...]) + mlpb_ref[...],
                    0.0),
        lnw_ref[...], lnb_ref[...])
    fuse = _ln(h_node + h_feat, 1.0, 0.0)
    gi_r = _dot_bf16(fuse, wir_ref[...]) + bir_ref[...]
    gi_z = _dot_bf16(fuse, wiz_ref[...]) + biz_ref[...]
    gi_n = _dot_bf16(fuse, win_ref[...]) + bin_ref[...]
    h = jnp.zeros((B, HID), f32)
    for t in range(T):
        sl = slice(t * B, (t + 1) * B)
        h_r = _dot_bf16(h, whr_ref[...]) + bhr_ref[...]
        h_z = _dot_bf16(h, whz_ref[...]) + bhz_ref[...]
        h_n = _dot_bf16(h, whn_ref[...]) + bhn_ref[...]
        r = jax.nn.sigmoid(gi_r[sl, :] + h_r)
        z = jax.nn.sigmoid(gi_z[sl, :] + h_z)
        n = jnp.tanh(gi_n[sl, :] + r * h_n)
        h = (1.0 - z) * n + z * h
    o = jnp.maximum(_dot_bf16(h, l1w_ref[...]) + l1b_ref[...], 0.0)
    out_ref[...] = _dot_bf16(o, l2w_ref[...]) + l2b_ref[...]


def kernel(nbIp, edge_index, snap_feat, target_idx, W_gat, att_src, att_dst,
           gat_bias, mlp_w, mlp_b, ln_w, ln_b, W_ih, W_hh, b_ih, b_hh,
           l1_w, l1_b, l2_w, l2_b):
    src = edge_index[0]
    dst = edge_index[1]
    tgt = target_idx.reshape(-1)
    ab = _sc_gat(src, dst, nbIp, tgt, W_gat.reshape(-1), att_src.reshape(-1),
                 att_dst.reshape(-1))

    snap_t = jnp.transpose(snap_feat, (1, 0, 2)).reshape(G, FEAT)
    l2_pad = jnp.zeros((32, 128), jnp.float32).at[:, 0].set(l2_w[:, 0])
    l2b_pad = jnp.zeros((1, 128), jnp.float32).at[0, 0].set(l2_b[0])
    args = [
        ab, W_gat, gat_bias.reshape(1, HID), snap_t, mlp_w,
        mlp_b.reshape(1, HID), ln_w.reshape(1, HID), ln_b.reshape(1, HID),
        W_ih[0:HID, :].T, W_ih[HID:2 * HID, :].T, W_ih[2 * HID:, :].T,
        W_hh[0:HID, :].T, W_hh[HID:2 * HID, :].T, W_hh[2 * HID:, :].T,
        b_ih[0:HID].reshape(1, HID), b_ih[HID:2 * HID].reshape(1, HID),
        b_ih[2 * HID:].reshape(1, HID),
        b_hh[0:HID].reshape(1, HID), b_hh[HID:2 * HID].reshape(1, HID),
        b_hh[2 * HID:].reshape(1, HID),
        l1_w, l1_b.reshape(1, 32), l2_pad, l2b_pad,
    ]
    out = pl.pallas_call(
        _tc_head,
        out_shape=jax.ShapeDtypeStruct((B, 128), jnp.float32),
    )(*args)
    return out[:, 0:1]


# speedup vs baseline: 3194.8499x; 1.2264x over previous
import functools

import jax
import jax.numpy as jnp
from jax import lax
from jax.experimental import pallas as pl
from jax.experimental.pallas import tpu as pltpu
from jax.experimental.pallas import tpu_sc as plsc

B, T, NPG, EPG, HID, HEADS, FEAT = 8, 8, 1000, 16000, 64, 4, 16
G = B * T
N = G * NPG
E = G * EPG
OUTC = HID // HEADS
NCHUNK = EPG // 16
DEG_PAD = 1008
NEG_BIG = -1e30


def _leaky(x):
    return jnp.where(x > 0, x, 0.2 * x)


def _round_bf16_f32(x):
    u = plsc.bitcast(x, jnp.int32)
    r = u + jnp.int32(0x7FFF) + (lax.shift_right_logical(u, 16) & 1)
    return plsc.bitcast(r & jnp.int32(-65536), jnp.float32)


_sc_mesh = plsc.VectorSubcoreMesh(core_axis_name="c", subcore_axis_name="s")


@functools.partial(
    pl.kernel,
    mesh=_sc_mesh,
    out_type=jax.ShapeDtypeStruct((G, 16), jnp.float32),
    compiler_params=pltpu.CompilerParams(needs_layout_passes=False),
    scratch_types=[
        pltpu.VMEM((EPG,), jnp.int32),
        pltpu.VMEM((EPG,), jnp.int32),
        pltpu.VMEM((EPG,), jnp.int32),
        pltpu.VMEM((EPG,), jnp.int32),
        pltpu.VMEM((EPG + 16,), jnp.int32),
        pltpu.VMEM((DEG_PAD,), jnp.float32),
        pltpu.VMEM((NPG,), jnp.float32),
        pltpu.VMEM((NPG,), jnp.float32),
        pltpu.VMEM((G,), jnp.int32),
        pltpu.VMEM((2 * HID,), jnp.float32),
        pltpu.VMEM((HID,), jnp.float32),
        pltpu.VMEM((HID,), jnp.float32),
        pltpu.VMEM((16,), jnp.float32),
        pltpu.SemaphoreType.DMA,
        pltpu.SemaphoreType.DMA,
    ],
)
def _sc_gat(ei_hbm, nbip_hbm, tgt_hbm, wg_hbm, as_hbm, ad_hbm,
            out_hbm, src_a, dst_a, src_b, dst_b, list_v, deg_v, nb_a, nb_b,
            tgt_v, wg_v, as_v, ad_v, row_v, sem_a, sem_b):
    wid = lax.axis_index("s") * 2 + lax.axis_index("c")
    lanes = lax.broadcasted_iota(jnp.int32, (16,), 0)
    zeros16 = jnp.zeros((16,), jnp.float32)

    g_a = wid * 2
    g_b = wid * 2 + 1
    cps = []
    for g, sv, dv, nv, sem in ((g_a, src_a, dst_a, nb_a, sem_a),
                               (g_b, src_b, dst_b, nb_b, sem_b)):
        cps.append((
            pltpu.async_copy(ei_hbm.at[0, pl.ds(g * EPG, EPG)], sv, sem),
            pltpu.async_copy(ei_hbm.at[1, pl.ds(g * EPG, EPG)], dv, sem),
            pltpu.async_copy(nbip_hbm.at[pl.ds(g * NPG, NPG)], nv, sem),
        ))

    pltpu.sync_copy(tgt_hbm, tgt_v)
    pltpu.sync_copy(wg_hbm, wg_v)
    pltpu.sync_copy(as_hbm, as_v)
    pltpu.sync_copy(ad_hbm, ad_v)
    for z in range(2 * HID // 16):
        wg_v[pl.ds(z * 16, 16)] = _round_bf16_f32(wg_v[pl.ds(z * 16, 16)])

    for k, (g, src_v, dst_v, nb_v) in enumerate(
            ((g_a, src_a, dst_a, nb_a), (g_b, src_b, dst_b, nb_b))):
        base = g * NPG
        for cp in cps[k]:
            cp.wait()

        for z in range(DEG_PAD // 16):
            deg_v[pl.ds(z * 16, 16)] = zeros16

        base_splat = jnp.full((16,), base, jnp.int32)
        tgt_splat = plsc.load_gather(tgt_v, [jnp.full((16,), g, jnp.int32)])

        def edge_body(i, cnt_v):
            off = i * 16
            s_loc = src_v[pl.ds(off, 16)] - base_splat
            d = dst_v[pl.ds(off, 16)]
            occ, last = plsc.scan_count(s_loc)
            plsc.addupdate_scatter(deg_v, [s_loc],
                                   occ.astype(jnp.float32), mask=last)
            msk = d == tgt_splat
            pre = plsc.cumsum(msk.astype(jnp.int32))
            plsc.store_scatter(list_v, [cnt_v + pre - 1], s_loc, mask=msk)
            return cnt_v + plsc.all_reduce_population_count(msk)

        cnt_v = plsc.parallel_loop(
            0, NCHUNK, 1, unroll=8,
            carry=jnp.zeros((16,), jnp.int32))(edge_body)
        cnt = jnp.max(cnt_v)

        for z in range(62):
            nb_v[pl.ds(z * 16, 16)] = _round_bf16_f32(nb_v[pl.ds(z * 16, 16)])
        nb_v[pl.ds(NPG - 16, 16)] = _round_bf16_f32(nb_v[pl.ds(NPG - 16, 16)])
        for z in range(DEG_PAD // 16):
            deg_v[pl.ds(z * 16, 16)] = _round_bf16_f32(
                deg_v[pl.ds(z * 16, 16)])

        t_loc = tgt_splat - base_splat
        nb_t = plsc.load_gather(nb_v, [t_loc])
        dg_t = plsc.load_gather(deg_v, [t_loc])
        nchunks = (cnt + 15) // 16

        row = zeros16
        for h in range(HEADS):
            w0h = wg_v[pl.ds(h * OUTC, 16)]
            w1h = wg_v[pl.ds(HID + h * OUTC, 16)]
            ash = as_v[pl.ds(h * OUTC, 16)]
            adh = ad_v[pl.ds(h * OUTC, 16)]
            c0 = jnp.sum(w0h * ash)
            c1 = jnp.sum(w1h * ash)
            d0 = jnp.sum(w0h * adh)
            d1 = jnp.sum(w1h * adh)
            a_d = nb_t * d0 + dg_t * d1
            a_self = _leaky(nb_t * c0 + dg_t * c1 + a_d)

            def alpha_at(j):
                valid = (lanes + j * 16) < cnt
                idx = jnp.where(valid, list_v[pl.ds(j * 16, 16)], 0)
                nb = plsc.load_gather(nb_v, [idx])
                dg = plsc.load_gather(deg_v, [idx])
                return valid, nb, dg, _leaky(nb * c0 + dg * c1 + a_d)

            def max_body(j, mx):
                valid, _, _, a = alpha_at(j)
                return jnp.maximum(mx, jnp.where(valid, a, NEG_BIG))

            m = jnp.max(lax.fori_loop(0, nchunks, max_body, a_self))

            def sum_body(j, acc):
                sa, sb, ss = acc
                valid, nb, dg, a = alpha_at(j)
                e = jnp.where(valid, jnp.exp(a - m), 0.0)
                return (sa + e * nb, sb + e * dg, ss + e)

            sa, sb, ss = lax.fori_loop(
                0, nchunks, sum_body, (zeros16, zeros16, zeros16))
            e_self = jnp.exp(a_self - m)
            S_v = e_self + jnp.sum(ss)
            A_v = (e_self * nb_t + jnp.sum(sa)) / S_v
            B_v = (e_self * dg_t + jnp.sum(sb)) / S_v
            row = jnp.where(lanes == h, A_v, row)
            row = jnp.where(lanes == (HEADS + h), B_v, row)

        row_v[...] = row
        r = (g % T) * B + (g // T)
        pltpu.sync_copy(row_v, out_hbm.at[r])


def _ln(x, w, b, eps=1e-5):
    mu = jnp.mean(x, axis=-1, keepdims=True)
    var = jnp.mean((x - mu) ** 2, axis=-1, keepdims=True)
    return (x - mu) / jnp.sqrt(var + eps) * w + b


def _dot_bf16(a, b):
    return jnp.dot(a.astype(jnp.bfloat16), b.astype(jnp.bfloat16),
                   preferred_element_type=jnp.float32)


def _dot_bf16_t(a, b):
    return lax.dot_general(a.astype(jnp.bfloat16), b.astype(jnp.bfloat16),
                           (((1,), (1,)), ((), ())),
                           preferred_element_type=jnp.float32)


def _tc_head(ab_ref, wg_ref, gb_ref, snap_ref, mlpw_ref, mlpb_ref, lnw_ref,
             lnb_ref, wih_ref, whh_ref, bih_ref, bhh_ref,
             l1w_ref, l1b_ref, l2w_ref, l2b_ref, out_ref):
    f32 = jnp.float32
    wg = wg_ref[...]
    rows = lax.broadcasted_iota(jnp.int32, (16, HID), 0)
    colh = lax.broadcasted_iota(jnp.int32, (16, HID), 1) // OUTC
    sel_a = jnp.where(rows == colh, 1.0, 0.0).astype(f32)
    sel_b = jnp.where(rows == colh + HEADS, 1.0, 0.0).astype(f32)
    ab = ab_ref[...]
    a_exp = jnp.dot(ab, sel_a, preferred_element_type=f32,
                    precision=lax.Precision.HIGHEST)
    b_exp = jnp.dot(ab, sel_b, preferred_element_type=f32,
                    precision=lax.Precision.HIGHEST)
    wgb = wg.astype(jnp.bfloat16).astype(f32)
    w0 = jnp.broadcast_to(wgb[0:1, :], (G, HID))
    w1 = jnp.broadcast_to(wgb[1:2, :], (G, HID))
    h_node = jnp.maximum(a_exp * w0 + b_exp * w1 + gb_ref[...], 0.0)
    h_feat = _ln(
        jnp.maximum(_dot_bf16(snap_ref[...], mlpw_ref[...]) + mlpb_ref[...],
                    0.0),
        lnw_ref[...], lnb_ref[...])
    fuse = _ln(h_node + h_feat, 1.0, 0.0)
    wih = wih_ref[...]
    whh = whh_ref[...]
    bih = bih_ref[...]
    bhh = bhh_ref[...]
    gi_r = _dot_bf16_t(fuse, wih[0:HID, :]) + bih[:, 0:HID]
    gi_z = _dot_bf16_t(fuse, wih[HID:2 * HID, :]) + bih[:, HID:2 * HID]
    gi_n = _dot_bf16_t(fuse, wih[2 * HID:, :]) + bih[:, 2 * HID:]
    h = jnp.zeros((B, HID), f32)
    for t in range(T):
        sl = slice(t * B, (t + 1) * B)
        h_r = _dot_bf16_t(h, whh[0:HID, :]) + bhh[:, 0:HID]
        h_z = _dot_bf16_t(h, whh[HID:2 * HID, :]) + bhh[:, HID:2 * HID]
        h_n = _dot_bf16_t(h, whh[2 * HID:, :]) + bhh[:, 2 * HID:]
        r = jax.nn.sigmoid(gi_r[sl, :] + h_r)
        z = jax.nn.sigmoid(gi_z[sl, :] + h_z)
        n = jnp.tanh(gi_n[sl, :] + r * h_n)
        h = (1.0 - z) * n + z * h
    o = jnp.maximum(_dot_bf16(h, l1w_ref[...]) + l1b_ref[...], 0.0)
    out_ref[...] = _dot_bf16(o, l2w_ref[...]) + l2b_ref[...]


def kernel(nbIp, edge_index, snap_feat, target_idx, W_gat, att_src, att_dst,
           gat_bias, mlp_w, mlp_b, ln_w, ln_b, W_ih, W_hh, b_ih, b_hh,
           l1_w, l1_b, l2_w, l2_b):
    ab = _sc_gat(edge_index, nbIp, target_idx.reshape(-1),
                 W_gat.reshape(-1), att_src.reshape(-1), att_dst.reshape(-1))

    snap_t = jnp.transpose(snap_feat, (1, 0, 2)).reshape(G, FEAT)
    args = [
        ab, W_gat, gat_bias.reshape(1, HID), snap_t, mlp_w,
        mlp_b.reshape(1, HID), ln_w.reshape(1, HID), ln_b.reshape(1, HID),
        W_ih, W_hh, b_ih.reshape(1, 3 * HID), b_hh.reshape(1, 3 * HID),
        l1_w, l1_b.reshape(1, 32), l2_w, l2_b.reshape(1, 1),
    ]
    return pl.pallas_call(
        _tc_head,
        out_shape=jax.ShapeDtypeStruct((B, 1), jnp.float32),
    )(*args)
